# baseline (speedup 1.0000x reference)
import jax
import jax.numpy as jnp
from jax.experimental import pallas as pl
from jax.experimental.pallas import tpu as pltpu

_BN = 512
_D = 64
_DA = 68


def _hi_lo(n2):
    hi = n2.astype(jnp.bfloat16)
    lo = (n2 - hi.astype(jnp.float32)).astype(jnp.bfloat16)
    return hi, lo


def _dist_kernel(e_ref, cb_ref, o_ref, ca_ref):
    @pl.when(pl.program_id(0) == 0)
    def _():
        c = cb_ref[...]
        c2 = jnp.sum(c * c, axis=1, keepdims=True)
        hi, lo = _hi_lo(c2)
        one = jnp.ones_like(hi)
        ca_ref[...] = jnp.concatenate(
            [c.astype(jnp.bfloat16), one, one, hi, lo], axis=1).T

    x = e_ref[pl.ds(pl.program_id(0) * _BN, _BN), :]
    z2 = jnp.sum(x * x, axis=1, keepdims=True)
    hi, lo = _hi_lo(z2)
    one = jnp.ones_like(hi)
    ea = jnp.concatenate(
        [(-2.0 * x).astype(jnp.bfloat16), hi, lo, one, one], axis=1)
    o_ref[...] = jax.lax.dot_general(
        ea, ca_ref[...], (((1,), (0,)), ((), ())),
        preferred_element_type=jnp.float32,
    )


def kernel(encoding, codebook):
    n, d = encoding.shape
    k, _ = codebook.shape
    return pl.pallas_call(
        _dist_kernel,
        grid=(n // _BN,),
        in_specs=[
            pl.BlockSpec((n, d), lambda i: (0, 0)),
            pl.BlockSpec((k, d), lambda i: (0, 0)),
        ],
        out_specs=pl.BlockSpec((_BN, k), lambda i: (i, 0)),
        out_shape=jax.ShapeDtypeStruct((n, k), jnp.float32),
        scratch_shapes=[pltpu.VMEM((_DA, k), jnp.bfloat16)],
        compiler_params=pltpu.CompilerParams(
            dimension_semantics=("arbitrary",),
            vmem_limit_bytes=63 * 1024 * 1024,
        ),
    )(encoding, codebook)


# speedup vs baseline: 1.0011x; 1.0011x over previous
import jax
import jax.numpy as jnp
from jax.experimental import pallas as pl
from jax.experimental.pallas import tpu as pltpu

_BN = 256
_D = 64
_DA = 68


def _hi_lo(n2):
    hi = n2.astype(jnp.bfloat16)
    lo = (n2 - hi.astype(jnp.float32)).astype(jnp.bfloat16)
    return hi, lo


def _dist_kernel(e_ref, cb_ref, o_ref, ca_ref):
    @pl.when(pl.program_id(0) == 0)
    def _():
        c = cb_ref[...]
        c2 = jnp.sum(c * c, axis=1, keepdims=True)
        hi, lo = _hi_lo(c2)
        one = jnp.ones_like(hi)
        ca_ref[...] = jnp.concatenate(
            [c.astype(jnp.bfloat16), one, one, hi, lo], axis=1).T

    x = e_ref[pl.ds(pl.program_id(0) * _BN, _BN), :]
    z2 = jnp.sum(x * x, axis=1, keepdims=True)
    hi, lo = _hi_lo(z2)
    one = jnp.ones_like(hi)
    ea = jnp.concatenate(
        [(-2.0 * x).astype(jnp.bfloat16), hi, lo, one, one], axis=1)
    o_ref[...] = jax.lax.dot_general(
        ea, ca_ref[...], (((1,), (0,)), ((), ())),
        preferred_element_type=jnp.float32,
    )


def kernel(encoding, codebook):
    n, d = encoding.shape
    k, _ = codebook.shape
    return pl.pallas_call(
        _dist_kernel,
        grid=(n // _BN,),
        in_specs=[
            pl.BlockSpec((n, d), lambda i: (0, 0)),
            pl.BlockSpec((k, d), lambda i: (0, 0)),
        ],
        out_specs=pl.BlockSpec((_BN, k), lambda i: (i, 0)),
        out_shape=jax.ShapeDtypeStruct((n, k), jnp.float32),
        scratch_shapes=[pltpu.VMEM((_DA, k), jnp.bfloat16)],
        compiler_params=pltpu.CompilerParams(
            dimension_semantics=("arbitrary",),
            vmem_limit_bytes=63 * 1024 * 1024,
        ),
    )(encoding, codebook)


# speedup vs baseline: 1.0704x; 1.0692x over previous
import jax
import jax.numpy as jnp
from jax.experimental import pallas as pl
from jax.experimental.pallas import tpu as pltpu

_BN = 512
_DA = 68


def _hi_lo(n2):
    hi = n2.astype(jnp.bfloat16)
    lo = (n2 - hi.astype(jnp.float32)).astype(jnp.bfloat16)
    return hi, lo


def _aug_t(xt, scale):
    n2 = jnp.sum(xt * xt, axis=0, keepdims=True)
    hi, lo = _hi_lo(n2)
    one = jnp.ones_like(hi)
    return (scale * xt).astype(jnp.bfloat16), hi, lo, one


def _dist_kernel(et_ref, ct_ref, o_ref, ca_ref):
    @pl.when(pl.program_id(0) == 0)
    def _():
        cb, hi, lo, one = _aug_t(ct_ref[...], 1.0)
        ca_ref[...] = jnp.concatenate([cb, one, one, hi, lo], axis=0)

    eb, hi, lo, one = _aug_t(et_ref[...], -2.0)
    ea = jnp.concatenate([eb, hi, lo, one, one], axis=0)
    o_ref[...] = jax.lax.dot_general(
        ea, ca_ref[...], (((0,), (0,)), ((), ())),
        preferred_element_type=jnp.float32,
    )


def kernel(encoding, codebook):
    n, d = encoding.shape
    k, _ = codebook.shape
    return pl.pallas_call(
        _dist_kernel,
        grid=(n // _BN,),
        in_specs=[
            pl.BlockSpec((d, _BN), lambda i: (0, i)),
            pl.BlockSpec((d, k), lambda i: (0, 0)),
        ],
        out_specs=pl.BlockSpec((_BN, k), lambda i: (i, 0)),
        out_shape=jax.ShapeDtypeStruct((n, k), jnp.float32),
        scratch_shapes=[pltpu.VMEM((_DA, k), jnp.bfloat16)],
        compiler_params=pltpu.CompilerParams(
            dimension_semantics=("arbitrary",),
            vmem_limit_bytes=63 * 1024 * 1024,
        ),
    )(encoding.T, codebook.T)
